# Initial kernel scaffold; baseline (speedup 1.0000x reference)
import jax
import jax.numpy as jnp
from jax.experimental import pallas as pl


def kernel(x, centroids):
    raise NotImplementedError("write your pallas kernel here")


# speedup vs baseline: 1.4273x; 1.4273x over previous
import functools

import jax
import jax.numpy as jnp
from jax import lax
from jax.experimental import pallas as pl
from jax.experimental.pallas import tpu as pltpu
from jax.experimental.pallas import tpu_sc as plsc

K = 1024
F = 256
N = 4096

NB = 512
NBLK = N // NB

NC = 2
NS = 16
NW = NC * NS
PW = N // NW
RW = K // NS
CW = 16


def _assign_body(c_ref, x_ref, nearest_ref, colsum_ref):
    C = c_ref[...]
    Xb = x_ref[...]
    c2 = jnp.sum(C * C, axis=1, keepdims=True)
    x2 = jnp.sum(Xb * Xb, axis=1)[None, :]
    prod = lax.dot_general(C, Xb, (((1,), (1,)), ((), ())),
                           preferred_element_type=jnp.float32)
    d = c2 + x2 - 2.0 * prod
    mn = jnp.min(d, axis=0, keepdims=True)
    kio = lax.broadcasted_iota(jnp.int32, d.shape, 0)
    near = jnp.min(jnp.where(d == mn, kio, K), axis=0)
    nearest_ref[...] = near.reshape(1, 1, NB)
    colsum_ref[...] = jnp.sum(d, axis=0).reshape(1, 1, NB)


_assign = pl.pallas_call(
    _assign_body,
    grid=(NBLK,),
    in_specs=[
        pl.BlockSpec((K, F), lambda i: (0, 0)),
        pl.BlockSpec((NB, F), lambda i: (i, 0)),
    ],
    out_specs=[
        pl.BlockSpec((1, 1, NB), lambda i: (i, 0, 0)),
        pl.BlockSpec((1, 1, NB), lambda i: (i, 0, 0)),
    ],
    out_shape=[
        jax.ShapeDtypeStruct((NBLK, 1, NB), jnp.int32),
        jax.ShapeDtypeStruct((NBLK, 1, NB), jnp.float32),
    ],
)


NP = N // NC


def _segment_body(xt_hbm, idx_hbm, zc_hbm,
                  sums_hbm, cnts_hbm,
                  idx_v, xtv, acc_v, cacc_v):
    c = lax.axis_index("c")
    s = lax.axis_index("s")
    pltpu.sync_copy(zc_hbm, acc_v)
    pltpu.sync_copy(zc_hbm, cacc_v)
    pltpu.sync_copy(idx_hbm.at[pl.ds(c * NP, NP)], idx_v)
    pltpu.sync_copy(xt_hbm.at[pl.ds(s * CW, CW), pl.ds(c * NP, NP)], xtv)

    def sums_step(j, carry):
        rows = idx_v[pl.ds(j * 16, 16)] * CW
        for f in range(CW):
            vals = xtv[f, pl.ds(j * 16, 16)]
            plsc.addupdate_scatter(acc_v, [rows + f], vals)
        return carry

    lax.fori_loop(0, NP // 16, sums_step, 0)

    ones = jnp.full((16,), 1.0, jnp.float32)

    def cnt_step(j, carry):
        rows = idx_v[pl.ds(j * 16, 16)] * CW
        plsc.addupdate_scatter(cacc_v, [rows], ones)
        return carry

    lax.fori_loop(s * (PW // 16), (s + 1) * (PW // 16), cnt_step, 0)

    pltpu.sync_copy(acc_v, sums_hbm.at[c, s])
    pltpu.sync_copy(cacc_v, cnts_hbm.at[c, s])


@functools.cache
def _make_segment():
    return pl.kernel(
        _segment_body,
        mesh=plsc.VectorSubcoreMesh(core_axis_name="c", subcore_axis_name="s"),
        compiler_params=pltpu.CompilerParams(needs_layout_passes=False),
        out_type=[
            jax.ShapeDtypeStruct((NC, NS, K * CW), jnp.float32),
            jax.ShapeDtypeStruct((NC, NS, K * CW), jnp.float32),
        ],
        scratch_types=[
            pltpu.VMEM((NP,), jnp.int32),
            pltpu.VMEM((CW, NP), jnp.float32),
            pltpu.VMEM((K * CW,), jnp.float32),
            pltpu.VMEM((K * CW,), jnp.float32),
        ],
    )


def _finalize_body(c_ref, sums_ref, cnts_ref, colsum_ref, out1_ref, out2_ref):
    cnt = jnp.zeros((K, 1), jnp.float32)
    for c in range(NC):
        for s in range(NS):
            cnt = cnt + cnts_ref[c, s][:, 0:1]
    nonempty = cnt > 0.0
    safe = jnp.maximum(cnt, 1.0)
    deltas = jnp.zeros((K, 1), jnp.float32)
    for s in range(NS):
        sg = sums_ref[0, s] + sums_ref[1, s]
        Cg = c_ref[:, s * CW:(s + 1) * CW]
        ng = jnp.where(nonempty, sg / safe, Cg)
        deltas = deltas + jnp.sum(jnp.square(Cg - ng), axis=1, keepdims=True)
    sum_delta = jnp.sum(jnp.where(nonempty, deltas, 0.0))
    delta_k = jnp.sum(nonempty.astype(jnp.float32))
    avg = jnp.sum(cnt * colsum_ref[...]) / N
    out1_ref[0, 0] = sum_delta / delta_k
    out2_ref[0, 0] = avg


_finalize = pl.pallas_call(
    _finalize_body,
    out_specs=[
        pl.BlockSpec(memory_space=pltpu.SMEM),
        pl.BlockSpec(memory_space=pltpu.SMEM),
    ],
    out_shape=[
        jax.ShapeDtypeStruct((1, 1), jnp.float32),
        jax.ShapeDtypeStruct((1, 1), jnp.float32),
    ],
)


def kernel(x, centroids):
    nearest3, colsum3 = _assign(centroids, x)
    nearest = nearest3.reshape(N)
    zc = jnp.zeros((K * CW,), jnp.float32)
    sums_p, cnts_p = _make_segment()(x.T, nearest, zc)
    sums_p = sums_p.reshape(NC, NS, K, CW)
    cnts_p = cnts_p.reshape(NC, NS, K, CW)
    colsum_k = colsum3.reshape(N)[:K].reshape(K, 1)
    s1, s2 = _finalize(centroids, sums_p, cnts_p, colsum_k)
    return (s1[0, 0], s2[0, 0])


# speedup vs baseline: 1.5064x; 1.0554x over previous
import functools

import jax
import jax.numpy as jnp
from jax import lax
from jax.experimental import pallas as pl
from jax.experimental.pallas import tpu as pltpu
from jax.experimental.pallas import tpu_sc as plsc

K = 1024
F = 256
N = 4096

NB = 512
NBLK = N // NB

NC = 2
NS = 16
NW = NC * NS
PW = N // NW
RW = K // NS
CW = 16


def _assign_body(c_ref, x_ref, nearest_ref, colsum_ref):
    C = c_ref[...]
    Xb = x_ref[...]
    c2 = jnp.sum(C * C, axis=1, keepdims=True)
    x2 = jnp.sum(Xb * Xb, axis=1)[None, :]
    prod = lax.dot_general(C, Xb, (((1,), (1,)), ((), ())),
                           preferred_element_type=jnp.float32)
    d = c2 + x2 - 2.0 * prod
    mn = jnp.min(d, axis=0, keepdims=True)
    kio = lax.broadcasted_iota(jnp.int32, d.shape, 0)
    near = jnp.min(jnp.where(d == mn, kio, K), axis=0)
    nearest_ref[...] = near.reshape(1, 1, NB)
    colsum_ref[...] = jnp.sum(d, axis=0).reshape(1, 1, NB)


_assign = pl.pallas_call(
    _assign_body,
    grid=(NBLK,),
    in_specs=[
        pl.BlockSpec((K, F), lambda i: (0, 0)),
        pl.BlockSpec((NB, F), lambda i: (i, 0)),
    ],
    out_specs=[
        pl.BlockSpec((1, 1, NB), lambda i: (i, 0, 0)),
        pl.BlockSpec((1, 1, NB), lambda i: (i, 0, 0)),
    ],
    out_shape=[
        jax.ShapeDtypeStruct((NBLK, 1, NB), jnp.int32),
        jax.ShapeDtypeStruct((NBLK, 1, NB), jnp.float32),
    ],
)


NP = N // NC


def _segment_body(xt_hbm, idx_hbm, zc_hbm,
                  sums_hbm, cnts_hbm,
                  idx_v, xtv, acc_v, cacc_v, sem):
    c = lax.axis_index("c")
    s = lax.axis_index("s")
    cps = [
        pltpu.make_async_copy(zc_hbm, acc_v, sem),
        pltpu.make_async_copy(zc_hbm.at[pl.ds(0, K)], cacc_v, sem),
        pltpu.make_async_copy(idx_hbm.at[pl.ds(c * NP, NP)], idx_v, sem),
        pltpu.make_async_copy(
            xt_hbm.at[pl.ds(s * CW, CW), pl.ds(c * NP, NP)], xtv, sem),
    ]
    for cp in cps:
        cp.start()
    for cp in cps:
        cp.wait()

    def sums_step(j, carry):
        rows = idx_v[pl.ds(j * 16, 16)] * CW
        for f in range(CW):
            vals = xtv[f, pl.ds(j * 16, 16)]
            plsc.addupdate_scatter(acc_v, [rows + f], vals)
        return carry

    lax.fori_loop(0, NP // 16, sums_step, 0)

    ones = jnp.full((16,), 1.0, jnp.float32)

    def cnt_step(j, carry):
        rows = idx_v[pl.ds(j * 16, 16)]
        plsc.addupdate_scatter(cacc_v, [rows], ones)
        return carry

    lax.fori_loop(s * (PW // 16), (s + 1) * (PW // 16), cnt_step, 0)

    wps = [
        pltpu.make_async_copy(acc_v, sums_hbm.at[c, s], sem),
        pltpu.make_async_copy(cacc_v, cnts_hbm.at[c, s], sem),
    ]
    for wp in wps:
        wp.start()
    for wp in wps:
        wp.wait()


@functools.cache
def _make_segment():
    return pl.kernel(
        _segment_body,
        mesh=plsc.VectorSubcoreMesh(core_axis_name="c", subcore_axis_name="s"),
        compiler_params=pltpu.CompilerParams(
            needs_layout_passes=False, disable_bounds_checks=True),
        out_type=[
            jax.ShapeDtypeStruct((NC, NS, K * CW), jnp.float32),
            jax.ShapeDtypeStruct((NC, NS, K), jnp.float32),
        ],
        scratch_types=[
            pltpu.VMEM((NP,), jnp.int32),
            pltpu.VMEM((CW, NP), jnp.float32),
            pltpu.VMEM((K * CW,), jnp.float32),
            pltpu.VMEM((K,), jnp.float32),
            pltpu.SemaphoreType.DMA,
        ],
    )


def _finalize_body(c_ref, sums_ref, cnts_ref, colsum_ref, out1_ref, out2_ref):
    cnt = jnp.zeros((K, 1), jnp.float32)
    for c in range(NC):
        for s in range(NS):
            cnt = cnt + cnts_ref[c, s]
    nonempty = cnt > 0.0
    safe = jnp.maximum(cnt, 1.0)
    deltas = jnp.zeros((K, 1), jnp.float32)
    for s in range(NS):
        sg = sums_ref[0, s] + sums_ref[1, s]
        Cg = c_ref[:, s * CW:(s + 1) * CW]
        ng = jnp.where(nonempty, sg / safe, Cg)
        deltas = deltas + jnp.sum(jnp.square(Cg - ng), axis=1, keepdims=True)
    sum_delta = jnp.sum(jnp.where(nonempty, deltas, 0.0))
    delta_k = jnp.sum(nonempty.astype(jnp.float32))
    avg = jnp.sum(cnt * colsum_ref[...]) / N
    out1_ref[0, 0] = sum_delta / delta_k
    out2_ref[0, 0] = avg


_finalize = pl.pallas_call(
    _finalize_body,
    out_specs=[
        pl.BlockSpec(memory_space=pltpu.SMEM),
        pl.BlockSpec(memory_space=pltpu.SMEM),
    ],
    out_shape=[
        jax.ShapeDtypeStruct((1, 1), jnp.float32),
        jax.ShapeDtypeStruct((1, 1), jnp.float32),
    ],
)


def kernel(x, centroids):
    nearest3, colsum3 = _assign(centroids, x)
    nearest = nearest3.reshape(N)
    zc = jnp.zeros((K * CW,), jnp.float32)
    sums_p, cnts_p = _make_segment()(x.T, nearest, zc)
    sums_p = sums_p.reshape(NC, NS, K, CW)
    cnts_p = cnts_p.reshape(NC, NS, K, 1)
    colsum_k = colsum3.reshape(N)[:K].reshape(K, 1)
    s1, s2 = _finalize(centroids, sums_p, cnts_p, colsum_k)
    return (s1[0, 0], s2[0, 0])


# speedup vs baseline: 2.5505x; 1.6932x over previous
import functools

import jax
import jax.numpy as jnp
from jax import lax
from jax.experimental import pallas as pl
from jax.experimental.pallas import tpu as pltpu
from jax.experimental.pallas import tpu_sc as plsc

K = 1024
F = 256
N = 4096

NB = 512
NBLK = N // NB

NC = 2
NS = 16
NW = NC * NS
PW = N // NW
RW = K // NS
CW = 16


def _assign_body(c_ref, x_ref, nearest_ref, colsum_ref):
    C = c_ref[...]
    Xb = x_ref[...]
    c2 = jnp.sum(C * C, axis=1, keepdims=True)
    x2 = jnp.sum(Xb * Xb, axis=1)[None, :]
    prod = lax.dot_general(C, Xb, (((1,), (1,)), ((), ())),
                           preferred_element_type=jnp.float32)
    d = c2 + x2 - 2.0 * prod
    mn = jnp.min(d, axis=0, keepdims=True)
    kio = lax.broadcasted_iota(jnp.int32, d.shape, 0)
    near = jnp.min(jnp.where(d == mn, kio, K), axis=0)
    nearest_ref[...] = near.reshape(1, 1, NB)
    colsum_ref[...] = jnp.sum(d, axis=0).reshape(1, 1, NB)


_assign = pl.pallas_call(
    _assign_body,
    grid=(NBLK,),
    in_specs=[
        pl.BlockSpec((K, F), lambda i: (0, 0)),
        pl.BlockSpec((NB, F), lambda i: (i, 0)),
    ],
    out_specs=[
        pl.BlockSpec((1, 1, NB), lambda i: (i, 0, 0)),
        pl.BlockSpec((1, 1, NB), lambda i: (i, 0, 0)),
    ],
    out_shape=[
        jax.ShapeDtypeStruct((NBLK, 1, NB), jnp.int32),
        jax.ShapeDtypeStruct((NBLK, 1, NB), jnp.float32),
    ],
)


NP = N // NC


def _segment_body(xt_hbm, idx_hbm, zc_hbm,
                  sums_hbm, cnts_hbm,
                  idx_v, xtv, acc_v, cacc_v, sem):
    c = lax.axis_index("c")
    s = lax.axis_index("s")
    cps = [
        pltpu.make_async_copy(zc_hbm, acc_v, sem),
        pltpu.make_async_copy(zc_hbm.at[pl.ds(0, K)], cacc_v, sem),
        pltpu.make_async_copy(idx_hbm.at[pl.ds(c * NP, NP)], idx_v, sem),
        pltpu.make_async_copy(
            xt_hbm.at[pl.ds(s * CW, CW), pl.ds(c * NP, NP)], xtv, sem),
    ]
    for cp in cps:
        cp.start()
    for cp in cps:
        cp.wait()

    def sums_step(j, carry):
        rows = idx_v[pl.ds(j * 16, 16)]
        for f in range(CW):
            vals = xtv[f, pl.ds(j * 16, 16)]
            plsc.addupdate_scatter(acc_v, [rows + (f * K)], vals)
        return carry

    lax.fori_loop(0, NP // 16, sums_step, 0)

    ones = jnp.full((16,), 1.0, jnp.float32)

    def cnt_step(j, carry):
        rows = idx_v[pl.ds(j * 16, 16)]
        plsc.addupdate_scatter(cacc_v, [rows], ones)
        return carry

    lax.fori_loop(s * (PW // 16), (s + 1) * (PW // 16), cnt_step, 0)

    wps = [
        pltpu.make_async_copy(acc_v, sums_hbm.at[c, s], sem),
        pltpu.make_async_copy(cacc_v, cnts_hbm.at[c, s], sem),
    ]
    for wp in wps:
        wp.start()
    for wp in wps:
        wp.wait()


@functools.cache
def _make_segment():
    return pl.kernel(
        _segment_body,
        mesh=plsc.VectorSubcoreMesh(core_axis_name="c", subcore_axis_name="s"),
        compiler_params=pltpu.CompilerParams(
            needs_layout_passes=False, disable_bounds_checks=True),
        out_type=[
            jax.ShapeDtypeStruct((NC, NS, K * CW), jnp.float32),
            jax.ShapeDtypeStruct((NC, NS, K), jnp.float32),
        ],
        scratch_types=[
            pltpu.VMEM((NP,), jnp.int32),
            pltpu.VMEM((CW, NP), jnp.float32),
            pltpu.VMEM((K * CW,), jnp.float32),
            pltpu.VMEM((K,), jnp.float32),
            pltpu.SemaphoreType.DMA,
        ],
    )


def _finalize_body(ct_ref, sums_ref, cnts_ref, colsum_ref, out1_ref, out2_ref):
    cnt = jnp.zeros((1, K), jnp.float32)
    for c in range(NC):
        for s in range(NS):
            cnt = cnt + cnts_ref[c, s]
    nonempty = cnt > 0.0
    safe = jnp.maximum(cnt, 1.0)
    deltas = jnp.zeros((1, K), jnp.float32)
    for s in range(NS):
        sg = sums_ref[0, s] + sums_ref[1, s]
        Cg = ct_ref[s * CW:(s + 1) * CW, :]
        ng = jnp.where(nonempty, sg / safe, Cg)
        deltas = deltas + jnp.sum(jnp.square(Cg - ng), axis=0, keepdims=True)
    sum_delta = jnp.sum(jnp.where(nonempty, deltas, 0.0))
    delta_k = jnp.sum(nonempty.astype(jnp.float32))
    avg = jnp.sum(cnt * colsum_ref[...]) / N
    out1_ref[0, 0] = sum_delta / delta_k
    out2_ref[0, 0] = avg


_finalize = pl.pallas_call(
    _finalize_body,
    out_specs=[
        pl.BlockSpec(memory_space=pltpu.SMEM),
        pl.BlockSpec(memory_space=pltpu.SMEM),
    ],
    out_shape=[
        jax.ShapeDtypeStruct((1, 1), jnp.float32),
        jax.ShapeDtypeStruct((1, 1), jnp.float32),
    ],
)


def kernel(x, centroids):
    nearest3, colsum3 = _assign(centroids, x)
    nearest = nearest3.reshape(N)
    zc = jnp.zeros((K * CW,), jnp.float32)
    sums_p, cnts_p = _make_segment()(x.T, nearest, zc)
    sums_p = sums_p.reshape(NC, NS, CW, K)
    cnts_p = cnts_p.reshape(NC, NS, 1, K)
    colsum_k = colsum3.reshape(N)[:K].reshape(1, K)
    s1, s2 = _finalize(centroids.T, sums_p, cnts_p, colsum_k)
    return (s1[0, 0], s2[0, 0])


# speedup vs baseline: 2.7672x; 1.0849x over previous
import functools

import jax
import jax.numpy as jnp
from jax import lax
from jax.experimental import pallas as pl
from jax.experimental.pallas import tpu as pltpu
from jax.experimental.pallas import tpu_sc as plsc

K = 1024
F = 256
N = 4096

NB = 512
NBLK = N // NB

NC = 2
NS = 16
NW = NC * NS
PW = N // NW
RW = K // NS
CW = 16


def _assign_body(c_ref, x_ref, nearest_ref, colsum_ref):
    C = c_ref[...]
    Xb = x_ref[...]
    c2 = jnp.sum(C * C, axis=1, keepdims=True)
    x2 = jnp.sum(Xb * Xb, axis=1)[None, :]
    prod = lax.dot_general(C, Xb, (((1,), (1,)), ((), ())),
                           preferred_element_type=jnp.float32)
    d = c2 + x2 - 2.0 * prod
    mn = jnp.min(d, axis=0, keepdims=True)
    kio = lax.broadcasted_iota(jnp.int32, d.shape, 0)
    near = jnp.min(jnp.where(d == mn, kio, K), axis=0)
    nearest_ref[...] = near.reshape(1, 1, NB)
    colsum_ref[...] = jnp.sum(d, axis=0).reshape(1, 1, NB)


_assign = pl.pallas_call(
    _assign_body,
    grid=(NBLK,),
    in_specs=[
        pl.BlockSpec((K, F), lambda i: (0, 0)),
        pl.BlockSpec((NB, F), lambda i: (i, 0)),
    ],
    out_specs=[
        pl.BlockSpec((1, 1, NB), lambda i: (i, 0, 0)),
        pl.BlockSpec((1, 1, NB), lambda i: (i, 0, 0)),
    ],
    out_shape=[
        jax.ShapeDtypeStruct((NBLK, 1, NB), jnp.int32),
        jax.ShapeDtypeStruct((NBLK, 1, NB), jnp.float32),
    ],
)


NP = N // NC


def _segment_body(xt_hbm, idx_hbm, zc_hbm,
                  sums_hbm, cnts_hbm,
                  idx_v, xtv, acc_v, cacc_v, sem):
    c = lax.axis_index("c")
    s = lax.axis_index("s")
    cps = [
        pltpu.make_async_copy(zc_hbm, acc_v, sem),
        pltpu.make_async_copy(zc_hbm.at[pl.ds(0, K)], cacc_v, sem),
        pltpu.make_async_copy(idx_hbm.at[pl.ds(c * NP, NP)], idx_v, sem),
        pltpu.make_async_copy(
            xt_hbm.at[pl.ds(s * CW, CW), pl.ds(c * NP, NP)], xtv, sem),
    ]
    for cp in cps:
        cp.start()
    for cp in cps:
        cp.wait()

    def sums_step(j, carry):
        rows = idx_v[pl.ds(j * 16, 16)]

        @plsc.parallel_loop(0, CW, 1, unroll=CW)
        def fbody(f):
            vals = xtv[f, pl.ds(j * 16, 16)]
            plsc.addupdate_scatter(acc_v, [rows + f * K], vals)

        return carry

    lax.fori_loop(0, NP // 16, sums_step, 0)

    ones = jnp.full((16,), 1.0, jnp.float32)

    def cnt_step(j, carry):
        rows = idx_v[pl.ds(j * 16, 16)]
        plsc.addupdate_scatter(cacc_v, [rows], ones)
        return carry

    lax.fori_loop(s * (PW // 16), (s + 1) * (PW // 16), cnt_step, 0)

    wps = [
        pltpu.make_async_copy(acc_v, sums_hbm.at[c, s], sem),
        pltpu.make_async_copy(cacc_v, cnts_hbm.at[c, s], sem),
    ]
    for wp in wps:
        wp.start()
    for wp in wps:
        wp.wait()


@functools.cache
def _make_segment():
    return pl.kernel(
        _segment_body,
        mesh=plsc.VectorSubcoreMesh(core_axis_name="c", subcore_axis_name="s"),
        compiler_params=pltpu.CompilerParams(
            needs_layout_passes=False, disable_bounds_checks=True),
        out_type=[
            jax.ShapeDtypeStruct((NC, NS, K * CW), jnp.float32),
            jax.ShapeDtypeStruct((NC, NS, K), jnp.float32),
        ],
        scratch_types=[
            pltpu.VMEM((NP,), jnp.int32),
            pltpu.VMEM((CW, NP), jnp.float32),
            pltpu.VMEM((K * CW,), jnp.float32),
            pltpu.VMEM((K,), jnp.float32),
            pltpu.SemaphoreType.DMA,
        ],
    )


def _finalize_body(ct_ref, sums_ref, cnts_ref, colsum_ref, out1_ref, out2_ref):
    cnt = jnp.zeros((1, K), jnp.float32)
    for c in range(NC):
        for s in range(NS):
            cnt = cnt + cnts_ref[c, s]
    nonempty = cnt > 0.0
    safe = jnp.maximum(cnt, 1.0)
    deltas = jnp.zeros((1, K), jnp.float32)
    for s in range(NS):
        sg = sums_ref[0, s] + sums_ref[1, s]
        Cg = ct_ref[s * CW:(s + 1) * CW, :]
        ng = jnp.where(nonempty, sg / safe, Cg)
        deltas = deltas + jnp.sum(jnp.square(Cg - ng), axis=0, keepdims=True)
    sum_delta = jnp.sum(jnp.where(nonempty, deltas, 0.0))
    delta_k = jnp.sum(nonempty.astype(jnp.float32))
    avg = jnp.sum(cnt * colsum_ref[...]) / N
    out1_ref[0, 0] = sum_delta / delta_k
    out2_ref[0, 0] = avg


_finalize = pl.pallas_call(
    _finalize_body,
    out_specs=[
        pl.BlockSpec(memory_space=pltpu.SMEM),
        pl.BlockSpec(memory_space=pltpu.SMEM),
    ],
    out_shape=[
        jax.ShapeDtypeStruct((1, 1), jnp.float32),
        jax.ShapeDtypeStruct((1, 1), jnp.float32),
    ],
)


def kernel(x, centroids):
    nearest3, colsum3 = _assign(centroids, x)
    nearest = nearest3.reshape(N)
    zc = jnp.zeros((K * CW,), jnp.float32)
    sums_p, cnts_p = _make_segment()(x.T, nearest, zc)
    sums_p = sums_p.reshape(NC, NS, CW, K)
    cnts_p = cnts_p.reshape(NC, NS, 1, K)
    colsum_k = colsum3.reshape(N)[:K].reshape(1, K)
    s1, s2 = _finalize(centroids.T, sums_p, cnts_p, colsum_k)
    return (s1[0, 0], s2[0, 0])


# speedup vs baseline: 2.9745x; 1.0749x over previous
import functools

import jax
import jax.numpy as jnp
from jax import lax
from jax.experimental import pallas as pl
from jax.experimental.pallas import tpu as pltpu
from jax.experimental.pallas import tpu_sc as plsc

K = 1024
F = 256
N = 4096

NB = 512
NBLK = N // NB

NC = 2
NS = 16
NW = NC * NS
PW = N // NW
RW = K // NS
CW = 16


def _assign_body(c_ref, x_ref, nearest_ref, colsum_ref, xt_ref, ct_ref):
    C = c_ref[...]
    Xb = x_ref[...]
    xt_ref[...] = Xb.T
    @pl.when(pl.program_id(0) == 0)
    def _():
        ct_ref[...] = C.T
    c2 = jnp.sum(C * C, axis=1, keepdims=True)
    x2 = jnp.sum(Xb * Xb, axis=1)[None, :]
    prod = lax.dot_general(C, Xb, (((1,), (1,)), ((), ())),
                           preferred_element_type=jnp.float32)
    d = c2 + x2 - 2.0 * prod
    mn = jnp.min(d, axis=0, keepdims=True)
    kio = lax.broadcasted_iota(jnp.int32, d.shape, 0)
    near = jnp.min(jnp.where(d == mn, kio, K), axis=0)
    nearest_ref[...] = near.reshape(1, 1, NB)
    colsum_ref[...] = jnp.sum(d, axis=0).reshape(1, 1, NB)


_assign = pl.pallas_call(
    _assign_body,
    grid=(NBLK,),
    in_specs=[
        pl.BlockSpec((K, F), lambda i: (0, 0)),
        pl.BlockSpec((NB, F), lambda i: (i, 0)),
    ],
    out_specs=[
        pl.BlockSpec((1, 1, NB), lambda i: (i, 0, 0)),
        pl.BlockSpec((1, 1, NB), lambda i: (i, 0, 0)),
        pl.BlockSpec((F, NB), lambda i: (0, i)),
        pl.BlockSpec((F, K), lambda i: (0, 0)),
    ],
    out_shape=[
        jax.ShapeDtypeStruct((NBLK, 1, NB), jnp.int32),
        jax.ShapeDtypeStruct((NBLK, 1, NB), jnp.float32),
        jax.ShapeDtypeStruct((F, N), jnp.float32),
        jax.ShapeDtypeStruct((F, K), jnp.float32),
    ],
)


NP = N // NC


def _segment_body(xt_hbm, idx_hbm, zc_hbm,
                  sums_hbm, cnts_hbm,
                  idx_v, xtv, acc_v, cacc_v, sem):
    c = lax.axis_index("c")
    s = lax.axis_index("s")
    cps = [
        pltpu.make_async_copy(zc_hbm, acc_v, sem),
        pltpu.make_async_copy(zc_hbm.at[pl.ds(0, K)], cacc_v, sem),
        pltpu.make_async_copy(idx_hbm.at[pl.ds(c * NP, NP)], idx_v, sem),
        pltpu.make_async_copy(
            xt_hbm.at[pl.ds(s * CW, CW), pl.ds(c * NP, NP)], xtv, sem),
    ]
    for cp in cps:
        cp.start()
    for cp in cps:
        cp.wait()

    def sums_step(j, carry):
        rows = idx_v[pl.ds(j * 16, 16)]

        @plsc.parallel_loop(0, CW, 1, unroll=CW)
        def fbody(f):
            vals = xtv[f, pl.ds(j * 16, 16)]
            plsc.addupdate_scatter(acc_v, [rows + f * K], vals)

        return carry

    lax.fori_loop(0, NP // 16, sums_step, 0)

    ones = jnp.full((16,), 1.0, jnp.float32)

    def cnt_step(j, carry):
        rows = idx_v[pl.ds(j * 16, 16)]
        plsc.addupdate_scatter(cacc_v, [rows], ones)
        return carry

    lax.fori_loop(s * (PW // 16), (s + 1) * (PW // 16), cnt_step, 0)

    wps = [
        pltpu.make_async_copy(acc_v, sums_hbm.at[c, s], sem),
        pltpu.make_async_copy(cacc_v, cnts_hbm.at[c, s], sem),
    ]
    for wp in wps:
        wp.start()
    for wp in wps:
        wp.wait()


@functools.cache
def _make_segment():
    return pl.kernel(
        _segment_body,
        mesh=plsc.VectorSubcoreMesh(core_axis_name="c", subcore_axis_name="s"),
        compiler_params=pltpu.CompilerParams(
            needs_layout_passes=False, disable_bounds_checks=True),
        out_type=[
            jax.ShapeDtypeStruct((NC, NS, K * CW), jnp.float32),
            jax.ShapeDtypeStruct((NC, NS, K), jnp.float32),
        ],
        scratch_types=[
            pltpu.VMEM((NP,), jnp.int32),
            pltpu.VMEM((CW, NP), jnp.float32),
            pltpu.VMEM((K * CW,), jnp.float32),
            pltpu.VMEM((K,), jnp.float32),
            pltpu.SemaphoreType.DMA,
        ],
    )


def _finalize_body(ct_ref, sums_ref, cnts_ref, colsum_ref, out1_ref, out2_ref):
    cnt = jnp.zeros((1, K), jnp.float32)
    for c in range(NC):
        for s in range(NS):
            cnt = cnt + cnts_ref[c, s]
    nonempty = cnt > 0.0
    safe = jnp.maximum(cnt, 1.0)
    deltas = jnp.zeros((1, K), jnp.float32)
    for s in range(NS):
        sg = sums_ref[0, s] + sums_ref[1, s]
        Cg = ct_ref[s * CW:(s + 1) * CW, :]
        ng = jnp.where(nonempty, sg / safe, Cg)
        deltas = deltas + jnp.sum(jnp.square(Cg - ng), axis=0, keepdims=True)
    sum_delta = jnp.sum(jnp.where(nonempty, deltas, 0.0))
    delta_k = jnp.sum(nonempty.astype(jnp.float32))
    avg = jnp.sum(cnt * colsum_ref[...]) / N
    out1_ref[0, 0] = sum_delta / delta_k
    out2_ref[0, 0] = avg


_finalize = pl.pallas_call(
    _finalize_body,
    out_specs=[
        pl.BlockSpec(memory_space=pltpu.SMEM),
        pl.BlockSpec(memory_space=pltpu.SMEM),
    ],
    out_shape=[
        jax.ShapeDtypeStruct((1, 1), jnp.float32),
        jax.ShapeDtypeStruct((1, 1), jnp.float32),
    ],
)


def kernel(x, centroids):
    nearest3, colsum3, xt, ct = _assign(centroids, x)
    nearest = nearest3.reshape(N)
    zc = jnp.zeros((K * CW,), jnp.float32)
    sums_p, cnts_p = _make_segment()(xt, nearest, zc)
    sums_p = sums_p.reshape(NC, NS, CW, K)
    cnts_p = cnts_p.reshape(NC, NS, 1, K)
    colsum_k = colsum3.reshape(N)[:K].reshape(1, K)
    s1, s2 = _finalize(ct, sums_p, cnts_p, colsum_k)
    return (s1[0, 0], s2[0, 0])


# speedup vs baseline: 2.9900x; 1.0052x over previous
import functools

import jax
import jax.numpy as jnp
from jax import lax
from jax.experimental import pallas as pl
from jax.experimental.pallas import tpu as pltpu
from jax.experimental.pallas import tpu_sc as plsc

K = 1024
F = 256
N = 4096

NB = 512
NBLK = N // NB

NC = 2
NS = 16
NW = NC * NS
PW = N // NW
RW = K // NS
CW = 16


def _assign_body(c_ref, x_ref, nearest_ref, xt_ref, ct_ref):
    C = c_ref[...]
    Xb = x_ref[...]
    xt_ref[...] = Xb.T
    @pl.when(pl.program_id(0) == 0)
    def _():
        ct_ref[...] = C.T
    c2 = jnp.sum(C * C, axis=1, keepdims=True)
    prod = lax.dot_general(C, Xb, (((1,), (1,)), ((), ())),
                           preferred_element_type=jnp.float32)
    d = c2 - 2.0 * prod
    mn = jnp.min(d, axis=0, keepdims=True)
    kio = lax.broadcasted_iota(jnp.int32, d.shape, 0)
    near = jnp.min(jnp.where(d == mn, kio, K), axis=0)
    nearest_ref[...] = near.reshape(1, 1, NB)


_assign = pl.pallas_call(
    _assign_body,
    grid=(NBLK,),
    in_specs=[
        pl.BlockSpec((K, F), lambda i: (0, 0)),
        pl.BlockSpec((NB, F), lambda i: (i, 0)),
    ],
    out_specs=[
        pl.BlockSpec((1, 1, NB), lambda i: (i, 0, 0)),
        pl.BlockSpec((F, NB), lambda i: (0, i)),
        pl.BlockSpec((F, K), lambda i: (0, 0)),
    ],
    out_shape=[
        jax.ShapeDtypeStruct((NBLK, 1, NB), jnp.int32),
        jax.ShapeDtypeStruct((F, N), jnp.float32),
        jax.ShapeDtypeStruct((F, K), jnp.float32),
    ],
)


NP = N // NC


def _segment_body(xt_hbm, idx_hbm, zc_hbm,
                  sums_hbm, cnts_hbm,
                  idx_v, xtv, acc_v, cacc_v, sem):
    c = lax.axis_index("c")
    s = lax.axis_index("s")
    cps = [
        pltpu.make_async_copy(zc_hbm, acc_v, sem),
        pltpu.make_async_copy(zc_hbm.at[pl.ds(0, K)], cacc_v, sem),
        pltpu.make_async_copy(idx_hbm.at[pl.ds(c * NP, NP)], idx_v, sem),
        pltpu.make_async_copy(
            xt_hbm.at[pl.ds(s * CW, CW), pl.ds(c * NP, NP)], xtv, sem),
    ]
    for cp in cps:
        cp.start()
    for cp in cps:
        cp.wait()

    def sums_step(j, carry):
        rows = idx_v[pl.ds(j * 16, 16)]

        @plsc.parallel_loop(0, CW, 1, unroll=CW)
        def fbody(f):
            vals = xtv[f, pl.ds(j * 16, 16)]
            plsc.addupdate_scatter(acc_v, [rows + f * K], vals)

        return carry

    lax.fori_loop(0, NP // 16, sums_step, 0)

    ones = jnp.full((16,), 1.0, jnp.float32)

    def cnt_step(j, carry):
        rows = idx_v[pl.ds(j * 16, 16)]
        plsc.addupdate_scatter(cacc_v, [rows], ones)
        return carry

    lax.fori_loop(s * (PW // 16), (s + 1) * (PW // 16), cnt_step, 0)

    wps = [
        pltpu.make_async_copy(acc_v, sums_hbm.at[c, s], sem),
        pltpu.make_async_copy(cacc_v, cnts_hbm.at[c, s], sem),
    ]
    for wp in wps:
        wp.start()
    for wp in wps:
        wp.wait()


@functools.cache
def _make_segment():
    return pl.kernel(
        _segment_body,
        mesh=plsc.VectorSubcoreMesh(core_axis_name="c", subcore_axis_name="s"),
        compiler_params=pltpu.CompilerParams(
            needs_layout_passes=False, disable_bounds_checks=True),
        out_type=[
            jax.ShapeDtypeStruct((NC, NS, K * CW), jnp.float32),
            jax.ShapeDtypeStruct((NC, NS, K), jnp.float32),
        ],
        scratch_types=[
            pltpu.VMEM((NP,), jnp.int32),
            pltpu.VMEM((CW, NP), jnp.float32),
            pltpu.VMEM((K * CW,), jnp.float32),
            pltpu.VMEM((K,), jnp.float32),
            pltpu.SemaphoreType.DMA,
        ],
    )


def _finalize_body(ct_ref, sums_ref, cnts_ref, x1_ref, out1_ref, out2_ref):
    cnt = jnp.zeros((1, K), jnp.float32)
    for c in range(NC):
        for s in range(NS):
            cnt = cnt + cnts_ref[c, s]
    nonempty = cnt > 0.0
    safe = jnp.maximum(cnt, 1.0)
    deltas = jnp.zeros((1, K), jnp.float32)
    for s in range(NS):
        sg = sums_ref[0, s] + sums_ref[1, s]
        Cg = ct_ref[s * CW:(s + 1) * CW, :]
        ng = jnp.where(nonempty, sg / safe, Cg)
        deltas = deltas + jnp.sum(jnp.square(Cg - ng), axis=0, keepdims=True)
    sum_delta = jnp.sum(jnp.where(nonempty, deltas, 0.0))
    delta_k = jnp.sum(nonempty.astype(jnp.float32))
    ct = ct_ref[...]
    x1 = x1_ref[...]
    c2k = jnp.sum(ct * ct, axis=0, keepdims=True)
    s_c2 = jnp.sum(c2k)
    s_c = jnp.sum(ct, axis=1, keepdims=True)
    x2 = jnp.sum(x1 * x1, axis=1, keepdims=True)
    xs = lax.dot_general(x1, s_c, (((1,), (0,)), ((), ())),
                         preferred_element_type=jnp.float32)
    colsum = s_c2 + jnp.float32(K) * x2 - 2.0 * xs
    avg = lax.dot_general(cnt, colsum, (((1,), (0,)), ((), ())),
                          preferred_element_type=jnp.float32)[0, 0] / N
    out1_ref[0, 0] = sum_delta / delta_k
    out2_ref[0, 0] = avg


_finalize = pl.pallas_call(
    _finalize_body,
    grid=(1,),
    in_specs=[
        pl.BlockSpec((F, K), lambda i: (0, 0)),
        pl.BlockSpec((NC, NS, CW, K), lambda i: (0, 0, 0, 0)),
        pl.BlockSpec((NC, NS, 1, K), lambda i: (0, 0, 0, 0)),
        pl.BlockSpec((K, F), lambda i: (0, 0)),
    ],
    out_specs=[
        pl.BlockSpec((1, 1), lambda i: (0, 0), memory_space=pltpu.SMEM),
        pl.BlockSpec((1, 1), lambda i: (0, 0), memory_space=pltpu.SMEM),
    ],
    out_shape=[
        jax.ShapeDtypeStruct((1, 1), jnp.float32),
        jax.ShapeDtypeStruct((1, 1), jnp.float32),
    ],
)


def kernel(x, centroids):
    nearest3, xt, ct = _assign(centroids, x)
    nearest = nearest3.reshape(N)
    zc = jnp.zeros((K * CW,), jnp.float32)
    sums_p, cnts_p = _make_segment()(xt, nearest, zc)
    sums_p = sums_p.reshape(NC, NS, CW, K)
    cnts_p = cnts_p.reshape(NC, NS, 1, K)
    s1, s2 = _finalize(ct, sums_p, cnts_p, x)
    return (s1[0, 0], s2[0, 0])
